# baseline (speedup 1.0000x reference)
import functools

import jax
import jax.numpy as jnp
from jax import lax
from jax.experimental import pallas as pl
from jax.experimental.pallas import tpu as pltpu
from jax.experimental.pallas import tpu_sc as plsc

_NC = 2
_NS = 16
_NW = _NC * _NS


@functools.lru_cache(maxsize=None)
def _make_gather(B, D, chunk, nbuf):
    b_per_w = B // _NW
    n_chunks = b_per_w // chunk
    mesh = plsc.VectorSubcoreMesh(core_axis_name="c", subcore_axis_name="s")

    @functools.partial(
        pl.kernel,
        out_type=jax.ShapeDtypeStruct((B, D), jnp.float32),
        mesh=mesh,
        scratch_types=(
            [pltpu.VMEM((b_per_w,), jnp.int32)]
            + [pltpu.VMEM((chunk, D), jnp.float32)] * nbuf
            + [pltpu.SemaphoreType.DMA] * (2 * nbuf)
        ),
        compiler_params=pltpu.CompilerParams(use_tc_tiling_on_sc=False),
    )
    def gather_kernel(x_hbm, w_hbm, out_hbm, idx_v, *bufs_sems):
        rows = bufs_sems[:nbuf]
        gsem = bufs_sems[nbuf:2 * nbuf]
        wsem = bufs_sems[2 * nbuf:]
        wid = lax.axis_index("s") * _NC + lax.axis_index("c")
        base0 = wid * b_per_w

        pltpu.sync_copy(x_hbm.at[pl.ds(base0, b_per_w)], idx_v)

        def gather(i, b):
            return pltpu.async_copy(
                w_hbm.at[pl.ds(0, chunk)], rows[b], gsem[b])

        def writeback(i, b):
            return pltpu.async_copy(
                rows[b], out_hbm.at[pl.ds(base0 + i * chunk, chunk)], wsem[b])

        pending_w = [None] * nbuf
        for i in range(min(nbuf - 1, n_chunks)):
            gather(i, i % nbuf)
        for i in range(n_chunks):
            b = i % nbuf
            j = i + nbuf - 1
            if j < n_chunks:
                bj = j % nbuf
                if pending_w[bj] is not None:
                    pending_w[bj].wait()
                    pending_w[bj] = None
                gather(j, bj)
            pltpu.make_async_copy(
                w_hbm.at[pl.ds(0, chunk)], rows[b],
                gsem[b]).wait()
            pending_w[b] = writeback(i, b)
        for b in range(nbuf):
            if pending_w[b] is not None:
                pending_w[b].wait()

    return gather_kernel


def kernel(x, W):
    B0, H = x.shape
    V, D = W.shape
    B = B0 * H
    flat_x = x.reshape((B,)).astype(jnp.int32)
    out = _make_gather(B, D, 800, 4)(flat_x, W)
    return out.reshape((B0, H, D))


# speedup vs baseline: 1.1335x; 1.1335x over previous
import functools

import jax
import jax.numpy as jnp
from jax import lax
from jax.experimental import pallas as pl
from jax.experimental.pallas import tpu as pltpu
from jax.experimental.pallas import tpu_sc as plsc

_NC = 2
_NS = 16
_NW = _NC * _NS


@functools.lru_cache(maxsize=None)
def _make_gather(B, D, chunk, nbuf):
    b_per_w = B // _NW
    n_chunks = b_per_w // chunk
    mesh = plsc.VectorSubcoreMesh(core_axis_name="c", subcore_axis_name="s")

    @functools.partial(
        pl.kernel,
        out_type=jax.ShapeDtypeStruct((B, D), jnp.float32),
        mesh=mesh,
        scratch_types=(
            [pltpu.VMEM((chunk,), jnp.int32)] * nbuf
            + [pltpu.VMEM((chunk, D), jnp.float32)] * nbuf
            + [pltpu.SemaphoreType.DMA] * (3 * nbuf)
        ),
        compiler_params=pltpu.CompilerParams(use_tc_tiling_on_sc=False),
    )
    def gather_kernel(x_hbm, w_hbm, out_hbm, *scratch):
        idxb = scratch[:nbuf]
        rows = scratch[nbuf:2 * nbuf]
        isem = scratch[2 * nbuf:3 * nbuf]
        gsem = scratch[3 * nbuf:4 * nbuf]
        wsem = scratch[4 * nbuf:]
        wid = lax.axis_index("s") * _NC + lax.axis_index("c")
        base0 = wid * b_per_w

        def idx_copy(j, b):
            return pltpu.async_copy(
                x_hbm.at[pl.ds(base0 + j * chunk, chunk)], idxb[b], isem[b])

        def gather(j, b):
            return pltpu.async_copy(w_hbm.at[idxb[b]], rows[b], gsem[b])

        def writeback(i, b):
            return pltpu.async_copy(
                rows[b], out_hbm.at[pl.ds(base0 + i * chunk, chunk)], wsem[b])

        for j in range(min(nbuf, n_chunks)):
            idx_copy(j, j)
        for j in range(min(nbuf - 1, n_chunks)):
            pltpu.make_async_copy(
                x_hbm.at[pl.ds(base0 + j * chunk, chunk)], idxb[j],
                isem[j]).wait()
            gather(j, j)

        pending_w = [None] * nbuf
        for i in range(n_chunks):
            b = i % nbuf
            j = i + nbuf - 1
            if j < n_chunks:
                bj = j % nbuf
                if pending_w[bj] is not None:
                    pending_w[bj].wait()
                    pending_w[bj] = None
                pltpu.make_async_copy(
                    x_hbm.at[pl.ds(base0 + j * chunk, chunk)], idxb[bj],
                    isem[bj]).wait()
                gather(j, bj)
            pltpu.make_async_copy(
                w_hbm.at[idxb[b]], rows[b], gsem[b]).wait()
            pending_w[b] = writeback(i, b)
            if i + nbuf < n_chunks:
                idx_copy(i + nbuf, b)
        for b in range(nbuf):
            if pending_w[b] is not None:
                pending_w[b].wait()

    return gather_kernel


def kernel(x, W):
    B0, H = x.shape
    V, D = W.shape
    B = B0 * H
    flat_x = x.reshape((B,)).astype(jnp.int32)
    out = _make_gather(B, D, 800, 4)(flat_x, W)
    return out.reshape((B0, H, D))
